# baseline (speedup 1.0000x reference)
import jax
import jax.numpy as jnp
from jax import lax
from jax.experimental import pallas as pl
from jax.experimental.pallas import tpu as pltpu

_H = _W = 32
_CIN, _K, _OC1, _OC2, _FC1 = 3, 5, 5, 10, 32
_OH1, _P1H, _OH2, _P2H = 28, 14, 10, 5

_B = 512
_NPIX = _CIN * _H * _W
_XROWS = 3136
_K1 = 512
_M1 = _OC1 * 32
_P1ROWS = 9 * 80 + _K1
_K2 = 512
_M2 = _OC2 * 16
_P2ROWS = 512


def _hi_lo(v):
    hi = v.astype(jnp.bfloat16)
    return hi, (v - hi.astype(jnp.float32)).astype(jnp.bfloat16)


def _body(x_ref, w1m, w1ml, b1m, w2m, w2ml, b2m, w1p, w1pl, b1fc,
          w2fc, w2fcl, b2fc, out_ref, xsh, xsl, p1h, p1l, p2h, p2l):
    f32 = jnp.float32
    bf16 = jnp.bfloat16

    def xpose(j):
        ic, h0 = j // 8, (j % 8) * 4
        xt = x_ref[:, j * 128:(j + 1) * 128].T
        hi, lo = _hi_lo(xt)
        for t in range(4):
            r = pl.ds((h0 + t) * 96 + ic * 32, 32)
            xsh[r, :] = hi[t * 32:(t + 1) * 32]
            xsl[r, :] = lo[t * 32:(t + 1) * 32]

    for ref in (xsh, xsl):
        ref[pl.ds(_NPIX, _XROWS - _NPIX), :] = jnp.zeros(
            (_XROWS - _NPIX, _B), bf16)
    for ref in (p1h, p1l):
        ref[pl.ds(14 * 80, _P1ROWS - 14 * 80), :] = jnp.zeros(
            (_P1ROWS - 14 * 80, _B), bf16)
    for ref in (p2h, p2l):
        ref[pl.ds(400, _P2ROWS - 400), :] = jnp.zeros(
            (_P2ROWS - 400, _B), bf16)

    for j in (0, 1, 2, 8, 9, 10, 16, 17, 18):
        xpose(j)

    def dot3(w, wl, hi_ref, lo_ref, r, k):
        hi = hi_ref[pl.ds(r, k), :]
        return (jnp.dot(w[...], hi, preferred_element_type=f32)
                + jnp.dot(w[...], lo_ref[pl.ds(r, k), :],
                          preferred_element_type=f32)
                + jnp.dot(wl[...], hi, preferred_element_type=f32))

    for ph in range(_P1H):
        if ph < 5:
            for ic in range(_CIN):
                xpose(ic * 8 + 3 + ph)
        a = dot3(w1m, w1ml, xsh, xsl, 2 * ph * 96, _K1).reshape(
            _OC1, 2, 16, _B)
        b = dot3(w1m, w1ml, xsh, xsl, (2 * ph + 1) * 96, _K1).reshape(
            _OC1, 2, 16, _B)
        m = jnp.maximum(jnp.maximum(a[:, 0], a[:, 1]),
                        jnp.maximum(b[:, 0], b[:, 1])).reshape(80, _B)
        hi, lo = _hi_lo(jnp.maximum(m + b1m[...], 0.0))
        p1h[pl.ds(ph * 80, 80), :] = hi
        p1l[pl.ds(ph * 80, 80), :] = lo

    for q in range(_P2H):
        a = dot3(w2m, w2ml, p1h, p1l, 2 * q * 80, _K2).reshape(
            _OC2, 2, 8, _B)
        b = dot3(w2m, w2ml, p1h, p1l, (2 * q + 1) * 80, _K2).reshape(
            _OC2, 2, 8, _B)
        m = jnp.maximum(jnp.maximum(a[:, 0], a[:, 1]),
                        jnp.maximum(b[:, 0], b[:, 1])).reshape(80, _B)
        hi, lo = _hi_lo(jnp.maximum(m + b2m[...], 0.0))
        p2h[pl.ds(q * 80, 80), :] = hi
        p2l[pl.ds(q * 80, 80), :] = lo

    hfc = jnp.maximum(
        dot3(w1p, w1pl, p2h, p2l, 0, _P2ROWS) + b1fc[...], 0.0)
    fh, fl = _hi_lo(hfc)
    out_ref[...] = (
        jnp.dot(w2fc[...], fh, preferred_element_type=f32)
        + jnp.dot(w2fc[...], fl, preferred_element_type=f32)
        + jnp.dot(w2fcl[...], fh, preferred_element_type=f32)
        + b2fc[...])


def _stride2_toeplitz(w, rows, width):
    pitch = width + 2
    lead = w.shape[:-1]
    p = jnp.pad(w, [(0, 0)] * len(lead) + [(0, pitch - w.shape[-1])])
    p = jnp.broadcast_to(p[..., None, :], lead + (rows, pitch))
    p = p.reshape(lead + (rows * pitch,))[..., :rows * width]
    return p.reshape(lead + (rows, width))


def _conv1_matrix(w1):
    w1r = w1.reshape(_OC1, _CIN, _K, _K)
    evn = _stride2_toeplitz(w1r, 16, 32)
    odd = _stride2_toeplitz(jnp.pad(w1r, ((0, 0),) * 3 + ((1, 0),)), 16, 32)
    wm = jnp.concatenate([evn, odd], axis=3)
    wm = wm.transpose(0, 3, 2, 1, 4).reshape(_M1, 480)
    return jnp.pad(wm, ((0, 0), (0, _K1 - 480)))


def _conv2_matrix(w2):
    w2r = w2.reshape(_OC2, _OC1, _K, _K)
    evn = _stride2_toeplitz(w2r, 8, 16)
    odd = _stride2_toeplitz(jnp.pad(w2r, ((0, 0),) * 3 + ((1, 0),)), 8, 16)
    wm = jnp.concatenate([evn, odd], axis=3)
    wm = wm.transpose(0, 3, 2, 1, 4).reshape(_M2, 400)
    return jnp.pad(wm, ((0, 0), (0, _K2 - 400)))


def _fc1_matrix(w1u):
    w3 = jnp.pad(w1u.reshape(_FC1, _OC2, 280), ((0, 0), (0, 0), (0, 40)))
    w4 = w3.reshape(_FC1, _OC2, _P2H, 64)[:, :, :, 0:20:4]
    w4 = jnp.pad(w4.transpose(0, 2, 1, 3), ((0, 0),) * 3 + ((0, 3),))
    return jnp.pad(w4.reshape(_FC1, 400), ((0, 0), (0, _P2ROWS - 400)))


def kernel(w1, b1, w2, b2, w1u, b1fc, w2fc, b2fc, x_nchw):
    n = x_nchw.shape[0]
    g = pl.cdiv(n, _B)
    npad = g * _B

    x = x_nchw.astype(jnp.float32)
    if npad != n:
        x = jnp.pad(x, ((0, npad - n), (0, 0), (0, 0), (0, 0)))
    x2d = x.reshape(npad, _NPIX)

    bf16 = jnp.bfloat16
    w1f = _conv1_matrix(w1)
    w1m = w1f.astype(bf16)
    w1ml = (w1f - w1m.astype(jnp.float32)).astype(bf16)
    w2f = _conv2_matrix(w2)
    w2m = w2f.astype(bf16)
    w2ml = (w2f - w2m.astype(jnp.float32)).astype(bf16)
    w1p = _fc1_matrix(w1u)
    w1ph = w1p.astype(bf16)
    w1pl = (w1p - w1ph.astype(jnp.float32)).astype(bf16)
    w2fch = w2fc.astype(bf16)
    w2fcl = (w2fc - w2fch.astype(jnp.float32)).astype(bf16)
    b1m = jnp.repeat(b1.astype(jnp.float32), 16)[:, None]
    b2m = jnp.repeat(b2.astype(jnp.float32), 8)[:, None]

    fix = lambda s: pl.BlockSpec(s, lambda b: (0,) * len(s))

    out = pl.pallas_call(
        _body,
        out_shape=jax.ShapeDtypeStruct((1, npad), jnp.float32),
        grid=(g,),
        in_specs=[
            pl.BlockSpec((_B, _NPIX), lambda b: (b, 0)),
            fix((_M1, _K1)), fix((_M1, _K1)), fix((80, 1)),
            fix((_M2, _K2)), fix((_M2, _K2)), fix((80, 1)),
            fix((_FC1, _P2ROWS)), fix((_FC1, _P2ROWS)), fix((_FC1, 1)),
            fix((1, _FC1)), fix((1, _FC1)), fix((1, 1)),
        ],
        out_specs=pl.BlockSpec((1, _B), lambda b: (0, b)),
        scratch_shapes=[
            pltpu.VMEM((_XROWS, _B), bf16),
            pltpu.VMEM((_XROWS, _B), bf16),
            pltpu.VMEM((_P1ROWS, _B), bf16),
            pltpu.VMEM((_P1ROWS, _B), bf16),
            pltpu.VMEM((_P2ROWS, _B), bf16),
            pltpu.VMEM((_P2ROWS, _B), bf16),
        ],
        compiler_params=pltpu.CompilerParams(
            dimension_semantics=("parallel",),
            vmem_limit_bytes=64 * 1024 * 1024),
    )(x2d, w1m, w1ml, b1m, w2m, w2ml, b2m, w1ph, w1pl, b1fc,
      w2fch, w2fcl, b2fc)

    return out[0, :n]
